# Initial kernel scaffold; baseline (speedup 1.0000x reference)
import jax
import jax.numpy as jnp
from jax.experimental import pallas as pl


def kernel(v, dist, dist_emb, edge_index, Wsf_w, Wsf_b, bn1_g, bn1_b, bn2_g, bn2_b, gru_w, gru_b):
    raise NotImplementedError("write your pallas kernel here")


# speedup vs baseline: 1.2218x; 1.2218x over previous
import functools
from math import pi as PI

import jax
import jax.numpy as jnp
from jax import lax
from jax.experimental import pallas as pl
from jax.experimental.pallas import tpu as pltpu
from jax.experimental.pallas import tpu_sc as plsc

N = 10000
E = 320000
H = 128
NG = 50
F = 2 * H
EPS = 1e-5
CUTOFF = 10.0

NC = 2
NS = 16
L = 16
NW = NC * NS
EPW = E // NW
CH = 80
NCH = EPW // CH
CH2 = 80
NCH2 = EPW // CH2
NPS = N // NS
ZR = 125

@functools.lru_cache(maxsize=1)
def _mesh():
    return plsc.VectorSubcoreMesh(
        core_axis_name="c", subcore_axis_name="s",
        num_cores=NC, num_subcores=NS)


def _softplus(x):
    return jnp.maximum(x, 0.0) + jnp.log(1.0 + jnp.exp(-jnp.abs(x)))


def _sigmoid(x):
    return 1.0 / (1.0 + jnp.exp(-x))


def _node_proj_body(v_ref, wit_ref, wjt_ref, bias_ref, a_ref, b_ref):
    v = v_ref[...]
    a_ref[...] = jnp.dot(v, wit_ref[...], preferred_element_type=jnp.float32) + bias_ref[...]
    b_ref[...] = jnp.dot(v, wjt_ref[...], preferred_element_type=jnp.float32)


def _node_proj(v, wit, wjt, bias):
    return pl.pallas_call(
        _node_proj_body,
        out_shape=[
            jax.ShapeDtypeStruct((N, F), jnp.float32),
            jax.ShapeDtypeStruct((N, F), jnp.float32),
        ],
    )(v, wit, wjt, bias)


PBLK = 4000


def _edge_proj_body(emb_ref, wet_ref, p_ref):
    p_ref[...] = jnp.dot(emb_ref[...], wet_ref[...], preferred_element_type=jnp.float32)


def _edge_proj(dist_emb, wet):
    grid = E // PBLK
    return pl.pallas_call(
        _edge_proj_body,
        grid=(grid,),
        in_specs=[
            pl.BlockSpec((PBLK, NG), lambda b: (b, 0)),
            pl.BlockSpec((NG, F), lambda b: (0, 0)),
        ],
        out_specs=pl.BlockSpec((PBLK, F), lambda b: (b, 0)),
        out_shape=jax.ShapeDtypeStruct((E, F), jnp.float32),
    )(dist_emb, wet)


def _gather_stats_body(a_hbm, b_hbm, p_hbm, ii_hbm, jj_hbm, y_hbm, st_hbm,
                       iv, jv, abuf, bbuf, pbuf, accv, sem_a, sem_b, sem_p):
    wid = lax.axis_index("s") * NC + lax.axis_index("c")
    base0 = wid * EPW

    for k in range(F // L):
        accv[0, pl.ds(k * L, L)] = jnp.zeros((L,), jnp.float32)
        accv[1, pl.ds(k * L, L)] = jnp.zeros((L,), jnp.float32)

    def chunk_body(it, carry):
        base = base0 + it * CH
        pltpu.sync_copy(ii_hbm.at[pl.ds(base, CH)], iv)
        pltpu.sync_copy(jj_hbm.at[pl.ds(base, CH)], jv)
        ha = pltpu.async_copy(a_hbm.at[iv], abuf, sem_a)
        hb = pltpu.async_copy(b_hbm.at[jv], bbuf, sem_b)
        hp = pltpu.async_copy(p_hbm.at[pl.ds(base, CH)], pbuf, sem_p)
        ha.wait()
        hb.wait()
        hp.wait()
        for k in range(F // L):
            sl = pl.ds(k * L, L)

            def row_body(r, c):
                s1, s2 = c
                yv = abuf[r, sl] + bbuf[r, sl] + pbuf[r, sl]
                abuf[r, sl] = yv
                return (s1 + yv, s2 + yv * yv)

            s1, s2 = lax.fori_loop(
                0, CH, row_body,
                (jnp.zeros((L,), jnp.float32), jnp.zeros((L,), jnp.float32)))
            accv[0, sl] += s1
            accv[1, sl] += s2
        pltpu.sync_copy(abuf, y_hbm.at[pl.ds(base, CH)])
        return carry

    lax.fori_loop(0, NCH, chunk_body, 0)
    pltpu.sync_copy(accv, st_hbm.at[wid])


@functools.lru_cache(maxsize=1)
def _gather_stats():
    return pl.kernel(
        _gather_stats_body,
        out_type=[
            jax.ShapeDtypeStruct((E, F), jnp.float32),
            jax.ShapeDtypeStruct((NW, 2, F), jnp.float32),
        ],
        mesh=_mesh(),
        scratch_types=[
            pltpu.VMEM((CH,), jnp.int32),
            pltpu.VMEM((CH,), jnp.int32),
            pltpu.VMEM((CH, F), jnp.float32),
            pltpu.VMEM((CH, F), jnp.float32),
            pltpu.VMEM((CH, F), jnp.float32),
            pltpu.VMEM((2, F), jnp.float32),
            pltpu.SemaphoreType.DMA,
            pltpu.SemaphoreType.DMA,
            pltpu.SemaphoreType.DMA,
        ],
    )


def _stats_body(st_ref, g_ref, b_ref, out_ref):
    st = st_ref[...]
    s1 = jnp.sum(st[:, 0, :], axis=0, keepdims=True)
    s2 = jnp.sum(st[:, 1, :], axis=0, keepdims=True)
    mu = s1 * (1.0 / E)
    var = s2 * (1.0 / E) - mu * mu
    s = g_ref[...] / jnp.sqrt(var + EPS)
    t = b_ref[...] - mu * s
    out_ref[...] = jnp.concatenate([s, t], axis=0)


def _stats_finalize(st, bn1_g, bn1_b):
    return pl.pallas_call(
        _stats_body,
        out_shape=jax.ShapeDtypeStruct((2, F), jnp.float32),
    )(st, bn1_g, bn1_b)


ABLK = 4000


def _activate_body(y_ref, sct_ref, dist_ref, m_ref):
    y = y_ref[...]
    s = sct_ref[0:1, :]
    t = sct_ref[1:2, :]
    yh = y * s + t
    c = _softplus(yh[:, :H])
    f = _sigmoid(yh[:, H:])
    cw = 0.5 * (jnp.cos(dist_ref[...] * (PI / CUTOFF)) + 1.0)
    m_ref[...] = c * f * cw


def _activate(y, sct, dist2d):
    grid = E // ABLK
    return pl.pallas_call(
        _activate_body,
        grid=(grid,),
        in_specs=[
            pl.BlockSpec((ABLK, F), lambda b: (b, 0)),
            pl.BlockSpec((2, F), lambda b: (0, 0)),
            pl.BlockSpec((ABLK, 1), lambda b: (b, 0)),
        ],
        out_specs=pl.BlockSpec((ABLK, H), lambda b: (b, 0)),
        out_shape=jax.ShapeDtypeStruct((E, H), jnp.float32),
    )(y, sct, dist2d)


def _scatter_body(m_hbm, ii_hbm, out_hbm, iv, mbuf, zbuf, acc_sh):
    cid = lax.axis_index("c")
    sid = lax.axis_index("s")
    wid = sid * NC + cid

    def zrow(r, c):
        for k in range(H // L):
            zbuf[r, pl.ds(k * L, L)] = jnp.zeros((L,), jnp.float32)
        return c

    lax.fori_loop(0, ZR, zrow, 0)

    def zcopy(q, c):
        pltpu.sync_copy(zbuf, acc_sh.at[pl.ds(sid * NPS + q * ZR, ZR)])
        return c

    lax.fori_loop(0, NPS // ZR, zcopy, 0)
    plsc.subcore_barrier()

    def chunk_body(it, carry):
        base = wid * EPW + it * CH2
        pltpu.sync_copy(ii_hbm.at[pl.ds(base, CH2)], iv)
        pltpu.sync_copy(m_hbm.at[pl.ds(base, CH2)], mbuf)
        pltpu.sync_copy(mbuf, acc_sh.at[iv], add=True)
        return carry

    lax.fori_loop(0, NCH2, chunk_body, 0)
    plsc.subcore_barrier()

    @pl.when(sid == 0)
    def _():
        pltpu.sync_copy(acc_sh, out_hbm.at[cid])


@functools.lru_cache(maxsize=1)
def _scatter():
    return pl.kernel(
        _scatter_body,
        out_type=jax.ShapeDtypeStruct((NC, N, H), jnp.float32),
        mesh=_mesh(),
        scratch_types=[
            pltpu.VMEM((CH2,), jnp.int32),
            pltpu.VMEM((CH2, H), jnp.float32),
            pltpu.VMEM((ZR, H), jnp.float32),
            pltpu.VMEM_SHARED((N, H), jnp.float32),
        ],
    )


def _final_body(aggp_ref, v_ref, g_ref, b_ref, wg1_ref, wg2_ref, gb_ref, out_ref):
    agg = aggp_ref[0] + aggp_ref[1]
    mu = jnp.mean(agg, axis=0, keepdims=True)
    var = jnp.mean(agg * agg, axis=0, keepdims=True) - mu * mu
    x2 = (agg - mu) / jnp.sqrt(var + EPS) * g_ref[...] + b_ref[...]
    v = v_ref[...]
    sg = _sigmoid(
        jnp.dot(v, wg1_ref[...], preferred_element_type=jnp.float32)
        + jnp.dot(x2, wg2_ref[...], preferred_element_type=jnp.float32)
        + gb_ref[...])
    out_ref[...] = _softplus(sg * v + (1.0 - sg) * x2)


def _final(aggp, v, bn2_g, bn2_b, wg1, wg2, gru_b):
    return pl.pallas_call(
        _final_body,
        out_shape=jax.ShapeDtypeStruct((N, H), jnp.float32),
    )(aggp, v, bn2_g, bn2_b, wg1, wg2, gru_b)


def kernel(v, dist, dist_emb, edge_index, Wsf_w, Wsf_b,
           bn1_g, bn1_b, bn2_g, bn2_b, gru_w, gru_b):
    i = edge_index[1].astype(jnp.int32)
    j = edge_index[0].astype(jnp.int32)
    wit = Wsf_w[:, :H].T
    wjt = Wsf_w[:, H:2 * H].T
    wet = Wsf_w[:, 2 * H:].T
    wg1 = gru_w[:, :H].T
    wg2 = gru_w[:, H:].T

    a, b = _node_proj(v, wit, wjt, Wsf_b[None, :])
    p = _edge_proj(dist_emb, wet)
    y, st = _gather_stats()(a, b, p, i, j)
    sct = _stats_finalize(st, bn1_g[None, :], bn1_b[None, :])
    m = _activate(y, sct, dist.reshape(E, 1))
    aggp = _scatter()(m, i)
    return _final(aggp, v, bn2_g[None, :], bn2_b[None, :], wg1, wg2,
                  gru_b[None, :])


# speedup vs baseline: 1.9770x; 1.6180x over previous
import functools
from math import pi as PI

import jax
import jax.numpy as jnp
from jax import lax
from jax.experimental import pallas as pl
from jax.experimental.pallas import tpu as pltpu
from jax.experimental.pallas import tpu_sc as plsc

N = 10000
E = 320000
H = 128
NG = 50
F = 2 * H
EPS = 1e-5
CUTOFF = 10.0

NC = 2
NS = 16
L = 16
NW = NC * NS
EPW = E // NW
CH = 80
NCH = EPW // CH
CH2 = 80
NCH2 = EPW // CH2
NPS = N // NS
ZR = 125

@functools.lru_cache(maxsize=1)
def _mesh():
    return plsc.VectorSubcoreMesh(
        core_axis_name="c", subcore_axis_name="s",
        num_cores=NC, num_subcores=NS)


def _softplus(x):
    return jnp.maximum(x, 0.0) + jnp.log(1.0 + jnp.exp(-jnp.abs(x)))


def _sigmoid(x):
    return 1.0 / (1.0 + jnp.exp(-x))


def _node_proj_body(v_ref, wit_ref, wjt_ref, bias_ref, a_ref, b_ref):
    v = v_ref[...]
    a_ref[...] = jnp.dot(v, wit_ref[...], preferred_element_type=jnp.float32) + bias_ref[...]
    b_ref[...] = jnp.dot(v, wjt_ref[...], preferred_element_type=jnp.float32)


def _node_proj(v, wit, wjt, bias):
    return pl.pallas_call(
        _node_proj_body,
        out_shape=[
            jax.ShapeDtypeStruct((N, F), jnp.float32),
            jax.ShapeDtypeStruct((N, F), jnp.float32),
        ],
    )(v, wit, wjt, bias)


PBLK = 4000


def _edge_proj_body(emb_ref, wet_ref, p_ref):
    p_ref[...] = jnp.dot(emb_ref[...], wet_ref[...], preferred_element_type=jnp.float32)


def _edge_proj(dist_emb, wet):
    grid = E // PBLK
    return pl.pallas_call(
        _edge_proj_body,
        grid=(grid,),
        in_specs=[
            pl.BlockSpec((PBLK, NG), lambda b: (b, 0)),
            pl.BlockSpec((NG, F), lambda b: (0, 0)),
        ],
        out_specs=pl.BlockSpec((PBLK, F), lambda b: (b, 0)),
        out_shape=jax.ShapeDtypeStruct((E, F), jnp.float32),
    )(dist_emb, wet)


def _gather_stats_body(a_hbm, b_hbm, p_hbm, ii_hbm, jj_hbm, y_hbm, st_hbm,
                       iv, jv, abuf, bbuf, pbuf, accv,
                       sem_i0, sem_i1, sem_g0, sem_g1, sem_y0, sem_y1):
    wid = lax.axis_index("s") * NC + lax.axis_index("c")
    base0 = wid * EPW
    sem_i = (sem_i0, sem_i1)
    sem_g = (sem_g0, sem_g1)
    sem_y = (sem_y0, sem_y1)

    for k in range(F // L):
        accv[0, pl.ds(k * L, L)] = jnp.zeros((L,), jnp.float32)
        accv[1, pl.ds(k * L, L)] = jnp.zeros((L,), jnp.float32)

    def idx_copies(c, b):
        base = base0 + c * CH
        return (pltpu.make_async_copy(ii_hbm.at[pl.ds(base, CH)], iv.at[b],
                                      sem_i[b]),
                pltpu.make_async_copy(jj_hbm.at[pl.ds(base, CH)], jv.at[b],
                                      sem_i[b]))

    def gat_copies(c, b):
        base = base0 + c * CH
        return (pltpu.make_async_copy(a_hbm.at[iv.at[b]], abuf.at[b],
                                      sem_g[b]),
                pltpu.make_async_copy(b_hbm.at[jv.at[b]], bbuf.at[b],
                                      sem_g[b]),
                pltpu.make_async_copy(p_hbm.at[pl.ds(base, CH)], pbuf.at[b],
                                      sem_g[b]))

    def y_copy(c, b):
        base = base0 + c * CH
        return pltpu.make_async_copy(abuf.at[b], y_hbm.at[pl.ds(base, CH)],
                                     sem_y[b])

    def start(descs):
        for d in (descs if isinstance(descs, tuple) else (descs,)):
            d.start()

    def wait(descs):
        for d in (descs if isinstance(descs, tuple) else (descs,)):
            d.wait()

    def compute(b):
        z = jnp.zeros((L,), jnp.float32)

        def row_body(r, acc):
            out = []
            for k in range(F // L):
                sl = pl.ds(k * L, L)
                yv = abuf[b, r, sl] + bbuf[b, r, sl] + pbuf[b, r, sl]
                abuf[b, r, sl] = yv
                out.append(acc[2 * k] + yv)
                out.append(acc[2 * k + 1] + yv * yv)
            return tuple(out)

        res = plsc.parallel_loop(0, CH, carry=(z,) * (2 * F // L))(row_body)
        for k in range(F // L):
            sl = pl.ds(k * L, L)
            accv[0, sl] += res[2 * k]
            accv[1, sl] += res[2 * k + 1]

    def iter_body(c, b):
        nb = 1 - b

        @pl.when(c + 1 < NCH)
        def _():
            @pl.when(c >= 1)
            def _():
                wait(y_copy(c - 1, nb))
            wait(idx_copies(c + 1, nb))
            start(gat_copies(c + 1, nb))

        wait(gat_copies(c, b))

        @pl.when(c + 2 < NCH)
        def _():
            start(idx_copies(c + 2, b))

        compute(b)
        start(y_copy(c, b))

    start(idx_copies(0, 0))
    wait(idx_copies(0, 0))
    start(gat_copies(0, 0))
    start(idx_copies(1, 1))

    def pair_body(p, carry):
        c0 = p * 2
        iter_body(c0, 0)
        iter_body(c0 + 1, 1)
        return carry

    lax.fori_loop(0, NCH // 2, pair_body, 0)
    if NCH % 2:
        iter_body(NCH - 1, 0)
    wait(y_copy(NCH - 2, 1 - (NCH - 1) % 2))
    wait(y_copy(NCH - 1, (NCH - 1) % 2))
    pltpu.sync_copy(accv, st_hbm.at[wid])


@functools.lru_cache(maxsize=1)
def _gather_stats():
    return pl.kernel(
        _gather_stats_body,
        out_type=[
            jax.ShapeDtypeStruct((E, F), jnp.float32),
            jax.ShapeDtypeStruct((NW, 2, F), jnp.float32),
        ],
        mesh=_mesh(),
        scratch_types=[
            pltpu.VMEM((2, CH), jnp.int32),
            pltpu.VMEM((2, CH), jnp.int32),
            pltpu.VMEM((2, CH, F), jnp.float32),
            pltpu.VMEM((2, CH, F), jnp.float32),
            pltpu.VMEM((2, CH, F), jnp.float32),
            pltpu.VMEM((2, F), jnp.float32),
            pltpu.SemaphoreType.DMA,
            pltpu.SemaphoreType.DMA,
            pltpu.SemaphoreType.DMA,
            pltpu.SemaphoreType.DMA,
            pltpu.SemaphoreType.DMA,
            pltpu.SemaphoreType.DMA,
        ],
    )


def _stats_body(st_ref, g_ref, b_ref, out_ref):
    st = st_ref[...]
    s1 = jnp.sum(st[:, 0, :], axis=0, keepdims=True)
    s2 = jnp.sum(st[:, 1, :], axis=0, keepdims=True)
    mu = s1 * (1.0 / E)
    var = s2 * (1.0 / E) - mu * mu
    s = g_ref[...] / jnp.sqrt(var + EPS)
    t = b_ref[...] - mu * s
    out_ref[...] = jnp.concatenate([s, t], axis=0)


def _stats_finalize(st, bn1_g, bn1_b):
    return pl.pallas_call(
        _stats_body,
        out_shape=jax.ShapeDtypeStruct((2, F), jnp.float32),
    )(st, bn1_g, bn1_b)


ABLK = 4000


def _activate_body(y_ref, sct_ref, dist_ref, m_ref):
    y = y_ref[...]
    s = sct_ref[0:1, :]
    t = sct_ref[1:2, :]
    yh = y * s + t
    c = _softplus(yh[:, :H])
    f = _sigmoid(yh[:, H:])
    cw = 0.5 * (jnp.cos(dist_ref[...] * (PI / CUTOFF)) + 1.0)
    m_ref[...] = c * f * cw


def _activate(y, sct, dist2d):
    grid = E // ABLK
    return pl.pallas_call(
        _activate_body,
        grid=(grid,),
        in_specs=[
            pl.BlockSpec((ABLK, F), lambda b: (b, 0)),
            pl.BlockSpec((2, F), lambda b: (0, 0)),
            pl.BlockSpec((ABLK, 1), lambda b: (b, 0)),
        ],
        out_specs=pl.BlockSpec((ABLK, H), lambda b: (b, 0)),
        out_shape=jax.ShapeDtypeStruct((E, H), jnp.float32),
    )(y, sct, dist2d)


def _scatter_body(m_hbm, ii_hbm, out_hbm, iv, mbuf, zbuf, acc_sh):
    cid = lax.axis_index("c")
    sid = lax.axis_index("s")
    wid = sid * NC + cid

    def zrow(r, c):
        for k in range(H // L):
            zbuf[r, pl.ds(k * L, L)] = jnp.zeros((L,), jnp.float32)
        return c

    lax.fori_loop(0, ZR, zrow, 0)

    def zcopy(q, c):
        pltpu.sync_copy(zbuf, acc_sh.at[pl.ds(sid * NPS + q * ZR, ZR)])
        return c

    lax.fori_loop(0, NPS // ZR, zcopy, 0)
    plsc.subcore_barrier()

    def chunk_body(it, carry):
        base = wid * EPW + it * CH2
        pltpu.sync_copy(ii_hbm.at[pl.ds(base, CH2)], iv)
        pltpu.sync_copy(m_hbm.at[pl.ds(base, CH2)], mbuf)
        pltpu.sync_copy(mbuf, acc_sh.at[iv], add=True)
        return carry

    lax.fori_loop(0, NCH2, chunk_body, 0)
    plsc.subcore_barrier()

    @pl.when(sid == 0)
    def _():
        pltpu.sync_copy(acc_sh, out_hbm.at[cid])


@functools.lru_cache(maxsize=1)
def _scatter():
    return pl.kernel(
        _scatter_body,
        out_type=jax.ShapeDtypeStruct((NC, N, H), jnp.float32),
        mesh=_mesh(),
        scratch_types=[
            pltpu.VMEM((CH2,), jnp.int32),
            pltpu.VMEM((CH2, H), jnp.float32),
            pltpu.VMEM((ZR, H), jnp.float32),
            pltpu.VMEM_SHARED((N, H), jnp.float32),
        ],
    )


def _final_body(aggp_ref, v_ref, g_ref, b_ref, wg1_ref, wg2_ref, gb_ref, out_ref):
    agg = aggp_ref[0] + aggp_ref[1]
    mu = jnp.mean(agg, axis=0, keepdims=True)
    var = jnp.mean(agg * agg, axis=0, keepdims=True) - mu * mu
    x2 = (agg - mu) / jnp.sqrt(var + EPS) * g_ref[...] + b_ref[...]
    v = v_ref[...]
    sg = _sigmoid(
        jnp.dot(v, wg1_ref[...], preferred_element_type=jnp.float32)
        + jnp.dot(x2, wg2_ref[...], preferred_element_type=jnp.float32)
        + gb_ref[...])
    out_ref[...] = _softplus(sg * v + (1.0 - sg) * x2)


def _final(aggp, v, bn2_g, bn2_b, wg1, wg2, gru_b):
    return pl.pallas_call(
        _final_body,
        out_shape=jax.ShapeDtypeStruct((N, H), jnp.float32),
    )(aggp, v, bn2_g, bn2_b, wg1, wg2, gru_b)


def kernel(v, dist, dist_emb, edge_index, Wsf_w, Wsf_b,
           bn1_g, bn1_b, bn2_g, bn2_b, gru_w, gru_b):
    i = edge_index[1].astype(jnp.int32)
    j = edge_index[0].astype(jnp.int32)
    wit = Wsf_w[:, :H].T
    wjt = Wsf_w[:, H:2 * H].T
    wet = Wsf_w[:, 2 * H:].T
    wg1 = gru_w[:, :H].T
    wg2 = gru_w[:, H:].T

    a, b = _node_proj(v, wit, wjt, Wsf_b[None, :])
    p = _edge_proj(dist_emb, wet)
    y, st = _gather_stats()(a, b, p, i, j)
    sct = _stats_finalize(st, bn1_g[None, :], bn1_b[None, :])
    m = _activate(y, sct, dist.reshape(E, 1))
    aggp = _scatter()(m, i)
    return _final(aggp, v, bn2_g[None, :], bn2_b[None, :], wg1, wg2,
                  gru_b[None, :])


# speedup vs baseline: 2.1449x; 1.0849x over previous
import functools
from math import pi as PI

import jax
import jax.numpy as jnp
from jax import lax
from jax.experimental import pallas as pl
from jax.experimental.pallas import tpu as pltpu
from jax.experimental.pallas import tpu_sc as plsc

N = 10000
E = 320000
H = 128
NG = 50
F = 2 * H
EPS = 1e-5
CUTOFF = 10.0

NC = 2
NS = 16
L = 16
NW = NC * NS
EPW = E // NW
CH = 80
NCH = EPW // CH
CH2 = 80
NCH2 = EPW // CH2
NPS = N // NS
ZR = 125

@functools.lru_cache(maxsize=1)
def _mesh():
    return plsc.VectorSubcoreMesh(
        core_axis_name="c", subcore_axis_name="s",
        num_cores=NC, num_subcores=NS)


def _softplus(x):
    return jnp.maximum(x, 0.0) + jnp.log(1.0 + jnp.exp(-jnp.abs(x)))


def _sigmoid(x):
    return 1.0 / (1.0 + jnp.exp(-x))


def _node_proj_body(v_ref, wit_ref, wjt_ref, bias_ref, a_ref, b_ref):
    v = v_ref[...]
    a_ref[...] = jnp.dot(v, wit_ref[...], preferred_element_type=jnp.float32) + bias_ref[...]
    b_ref[...] = jnp.dot(v, wjt_ref[...], preferred_element_type=jnp.float32)


def _node_proj(v, wit, wjt, bias):
    return pl.pallas_call(
        _node_proj_body,
        out_shape=[
            jax.ShapeDtypeStruct((N, F), jnp.float32),
            jax.ShapeDtypeStruct((N, F), jnp.float32),
        ],
    )(v, wit, wjt, bias)


PBLK = 4000


def _edge_proj_body(emb_ref, wet_ref, p_ref):
    p_ref[...] = jnp.dot(emb_ref[...], wet_ref[...], preferred_element_type=jnp.float32)


def _edge_proj(dist_emb, wet):
    grid = E // PBLK
    return pl.pallas_call(
        _edge_proj_body,
        grid=(grid,),
        in_specs=[
            pl.BlockSpec((PBLK, NG), lambda b: (b, 0)),
            pl.BlockSpec((NG, F), lambda b: (0, 0)),
        ],
        out_specs=pl.BlockSpec((PBLK, F), lambda b: (b, 0)),
        out_shape=jax.ShapeDtypeStruct((E, F), jnp.float32),
    )(dist_emb, wet)


def _gather_stats_body(a_hbm, b_hbm, p_hbm, ii_hbm, jj_hbm, y_hbm, st_hbm,
                       iv, jv, abuf, bbuf, pbuf, accv,
                       sem_i0, sem_i1, sem_g0, sem_g1, sem_y0, sem_y1):
    wid = lax.axis_index("s") * NC + lax.axis_index("c")
    base0 = wid * EPW
    sem_i = (sem_i0, sem_i1)
    sem_g = (sem_g0, sem_g1)
    sem_y = (sem_y0, sem_y1)

    for k in range(F // L):
        accv[0, pl.ds(k * L, L)] = jnp.zeros((L,), jnp.float32)
        accv[1, pl.ds(k * L, L)] = jnp.zeros((L,), jnp.float32)

    def idx_copies(c, b):
        base = base0 + c * CH
        return (pltpu.make_async_copy(ii_hbm.at[pl.ds(base, CH)], iv.at[b],
                                      sem_i[b]),
                pltpu.make_async_copy(jj_hbm.at[pl.ds(base, CH)], jv.at[b],
                                      sem_i[b]))

    def gat_copies(c, b):
        base = base0 + c * CH
        return (pltpu.make_async_copy(a_hbm.at[iv.at[b]], abuf.at[b],
                                      sem_g[b]),
                pltpu.make_async_copy(b_hbm.at[jv.at[b]], bbuf.at[b],
                                      sem_g[b]),
                pltpu.make_async_copy(p_hbm.at[pl.ds(base, CH)], pbuf.at[b],
                                      sem_g[b]))

    def y_copy(c, b):
        base = base0 + c * CH
        return pltpu.make_async_copy(abuf.at[b], y_hbm.at[pl.ds(base, CH)],
                                     sem_y[b])

    def start(descs):
        for d in (descs if isinstance(descs, tuple) else (descs,)):
            d.start()

    def wait(descs):
        for d in (descs if isinstance(descs, tuple) else (descs,)):
            d.wait()

    def compute(b):
        z = jnp.zeros((L,), jnp.float32)

        def row_body(r, acc):
            out = []
            for k in range(F // L):
                sl = pl.ds(k * L, L)
                yv = abuf[b, r, sl] + bbuf[b, r, sl] + pbuf[b, r, sl]
                abuf[b, r, sl] = yv
                out.append(acc[2 * k] + yv)
                out.append(acc[2 * k + 1] + yv * yv)
            return tuple(out)

        res = plsc.parallel_loop(0, CH, carry=(z,) * (2 * F // L))(row_body)
        for k in range(F // L):
            sl = pl.ds(k * L, L)
            accv[0, sl] += res[2 * k]
            accv[1, sl] += res[2 * k + 1]

    def iter_body(c, b):
        nb = 1 - b

        @pl.when(c + 1 < NCH)
        def _():
            @pl.when(c >= 1)
            def _():
                wait(y_copy(c - 1, nb))
            wait(idx_copies(c + 1, nb))
            start(gat_copies(c + 1, nb))

        wait(gat_copies(c, b))

        @pl.when(c + 2 < NCH)
        def _():
            start(idx_copies(c + 2, b))

        compute(b)
        start(y_copy(c, b))

    start(idx_copies(0, 0))
    wait(idx_copies(0, 0))
    start(gat_copies(0, 0))
    start(idx_copies(1, 1))

    def pair_body(p, carry):
        c0 = p * 2
        iter_body(c0, 0)
        iter_body(c0 + 1, 1)
        return carry

    lax.fori_loop(0, NCH // 2, pair_body, 0)
    if NCH % 2:
        iter_body(NCH - 1, 0)
    wait(y_copy(NCH - 2, 1 - (NCH - 1) % 2))
    wait(y_copy(NCH - 1, (NCH - 1) % 2))
    pltpu.sync_copy(accv, st_hbm.at[wid])


@functools.lru_cache(maxsize=1)
def _gather_stats():
    return pl.kernel(
        _gather_stats_body,
        out_type=[
            jax.ShapeDtypeStruct((E, F), jnp.float32),
            jax.ShapeDtypeStruct((NW, 2, F), jnp.float32),
        ],
        mesh=_mesh(),
        scratch_types=[
            pltpu.VMEM((2, CH), jnp.int32),
            pltpu.VMEM((2, CH), jnp.int32),
            pltpu.VMEM((2, CH, F), jnp.float32),
            pltpu.VMEM((2, CH, F), jnp.float32),
            pltpu.VMEM((2, CH, F), jnp.float32),
            pltpu.VMEM((2, F), jnp.float32),
            pltpu.SemaphoreType.DMA,
            pltpu.SemaphoreType.DMA,
            pltpu.SemaphoreType.DMA,
            pltpu.SemaphoreType.DMA,
            pltpu.SemaphoreType.DMA,
            pltpu.SemaphoreType.DMA,
        ],
    )


def _stats_body(st_ref, g_ref, b_ref, out_ref):
    st = st_ref[...]
    s1 = jnp.sum(st[:, 0, :], axis=0, keepdims=True)
    s2 = jnp.sum(st[:, 1, :], axis=0, keepdims=True)
    mu = s1 * (1.0 / E)
    var = s2 * (1.0 / E) - mu * mu
    s = g_ref[...] / jnp.sqrt(var + EPS)
    t = b_ref[...] - mu * s
    out_ref[...] = jnp.concatenate([s, t], axis=0)


def _stats_finalize(st, bn1_g, bn1_b):
    return pl.pallas_call(
        _stats_body,
        out_shape=jax.ShapeDtypeStruct((2, F), jnp.float32),
    )(st, bn1_g, bn1_b)


ABLK = 4000


def _activate_body(y_ref, sct_ref, dist_ref, m_ref):
    y = y_ref[...]
    s = sct_ref[0:1, :]
    t = sct_ref[1:2, :]
    yh = y * s + t
    c = _softplus(yh[:, :H])
    f = _sigmoid(yh[:, H:])
    cw = 0.5 * (jnp.cos(dist_ref[...] * (PI / CUTOFF)) + 1.0)
    m_ref[...] = c * f * cw


def _activate(y, sct, dist2d):
    grid = E // ABLK
    return pl.pallas_call(
        _activate_body,
        grid=(grid,),
        in_specs=[
            pl.BlockSpec((ABLK, F), lambda b: (b, 0)),
            pl.BlockSpec((2, F), lambda b: (0, 0)),
            pl.BlockSpec((ABLK, 1), lambda b: (b, 0)),
        ],
        out_specs=pl.BlockSpec((ABLK, H), lambda b: (b, 0)),
        out_shape=jax.ShapeDtypeStruct((E, H), jnp.float32),
    )(y, sct, dist2d)


def _scatter_body(m_hbm, ii_hbm, out_hbm, iv, mbuf, zbuf, acc_sh,
                  sem_l0, sem_l1, sem_s0, sem_s1):
    cid = lax.axis_index("c")
    sid = lax.axis_index("s")
    wid = sid * NC + cid
    sem_l = (sem_l0, sem_l1)
    sem_s = (sem_s0, sem_s1)

    def zrow(r, c):
        for k in range(H // L):
            zbuf[r, pl.ds(k * L, L)] = jnp.zeros((L,), jnp.float32)
        return c

    lax.fori_loop(0, ZR, zrow, 0)

    def zcopy(q, c):
        pltpu.sync_copy(zbuf, acc_sh.at[pl.ds(sid * NPS + q * ZR, ZR)])
        return c

    lax.fori_loop(0, NPS // ZR, zcopy, 0)
    plsc.subcore_barrier()

    def load_copies(c, b):
        base = wid * EPW + c * CH2
        return (pltpu.make_async_copy(ii_hbm.at[pl.ds(base, CH2)], iv.at[b],
                                      sem_l[b]),
                pltpu.make_async_copy(m_hbm.at[pl.ds(base, CH2)], mbuf.at[b],
                                      sem_l[b]))

    def start(descs):
        for d in (descs if isinstance(descs, tuple) else (descs,)):
            d.start()

    def wait(descs):
        for d in (descs if isinstance(descs, tuple) else (descs,)):
            d.wait()

    def scat_wait(b):
        pltpu.make_async_copy(mbuf.at[b], acc_sh.at[iv.at[b]], sem_s[b]).wait()

    def iter_body(c, b):
        nb = 1 - b

        @pl.when(c + 1 < NCH2)
        def _():
            @pl.when(c >= 1)
            def _():
                scat_wait(nb)
            start(load_copies(c + 1, nb))

        wait(load_copies(c, b))
        pltpu.async_copy(mbuf.at[b], acc_sh.at[iv.at[b]], sem_s[b], add=True)

    start(load_copies(0, 0))

    def pair_body(p, carry):
        c0 = p * 2
        iter_body(c0, 0)
        iter_body(c0 + 1, 1)
        return carry

    lax.fori_loop(0, NCH2 // 2, pair_body, 0)
    if NCH2 % 2:
        iter_body(NCH2 - 1, 0)
    scat_wait(1 - (NCH2 - 1) % 2)
    scat_wait((NCH2 - 1) % 2)
    plsc.subcore_barrier()

    @pl.when(sid == 0)
    def _():
        pltpu.sync_copy(acc_sh, out_hbm.at[cid])


@functools.lru_cache(maxsize=1)
def _scatter():
    return pl.kernel(
        _scatter_body,
        out_type=jax.ShapeDtypeStruct((NC, N, H), jnp.float32),
        mesh=_mesh(),
        scratch_types=[
            pltpu.VMEM((2, CH2), jnp.int32),
            pltpu.VMEM((2, CH2, H), jnp.float32),
            pltpu.VMEM((ZR, H), jnp.float32),
            pltpu.VMEM_SHARED((N, H), jnp.float32),
            pltpu.SemaphoreType.DMA,
            pltpu.SemaphoreType.DMA,
            pltpu.SemaphoreType.DMA,
            pltpu.SemaphoreType.DMA,
        ],
    )


def _final_body(aggp_ref, v_ref, g_ref, b_ref, wg1_ref, wg2_ref, gb_ref, out_ref):
    agg = aggp_ref[0] + aggp_ref[1]
    mu = jnp.mean(agg, axis=0, keepdims=True)
    var = jnp.mean(agg * agg, axis=0, keepdims=True) - mu * mu
    x2 = (agg - mu) / jnp.sqrt(var + EPS) * g_ref[...] + b_ref[...]
    v = v_ref[...]
    sg = _sigmoid(
        jnp.dot(v, wg1_ref[...], preferred_element_type=jnp.float32)
        + jnp.dot(x2, wg2_ref[...], preferred_element_type=jnp.float32)
        + gb_ref[...])
    out_ref[...] = _softplus(sg * v + (1.0 - sg) * x2)


def _final(aggp, v, bn2_g, bn2_b, wg1, wg2, gru_b):
    return pl.pallas_call(
        _final_body,
        out_shape=jax.ShapeDtypeStruct((N, H), jnp.float32),
    )(aggp, v, bn2_g, bn2_b, wg1, wg2, gru_b)


def kernel(v, dist, dist_emb, edge_index, Wsf_w, Wsf_b,
           bn1_g, bn1_b, bn2_g, bn2_b, gru_w, gru_b):
    i = edge_index[1].astype(jnp.int32)
    j = edge_index[0].astype(jnp.int32)
    wit = Wsf_w[:, :H].T
    wjt = Wsf_w[:, H:2 * H].T
    wet = Wsf_w[:, 2 * H:].T
    wg1 = gru_w[:, :H].T
    wg2 = gru_w[:, H:].T

    a, b = _node_proj(v, wit, wjt, Wsf_b[None, :])
    p = _edge_proj(dist_emb, wet)
    y, st = _gather_stats()(a, b, p, i, j)
    sct = _stats_finalize(st, bn1_g[None, :], bn1_b[None, :])
    m = _activate(y, sct, dist.reshape(E, 1))
    aggp = _scatter()(m, i)
    return _final(aggp, v, bn2_g[None, :], bn2_b[None, :], wg1, wg2,
                  gru_b[None, :])
